# baseline (speedup 1.0000x reference)
import jax
import jax.numpy as jnp
from jax.experimental import pallas as pl

BATCH = 256
N_CODES = 64
BOOK = 1024
D = 64

G = 8
B_BLK = 128

_DIST_PREC = jax.lax.Precision.DEFAULT
_GATHER_PREC = jax.lax.Precision.HIGHEST


def _vq_body(wq_ref, cb_ref, w_ref, we_ref, oh_ref):
    for g in range(G):
        x = wq_ref[:, g * D:(g + 1) * D]
        cb = cb_ref[g]
        xc = jax.lax.dot_general(
            x, cb, (((1,), (1,)), ((), ())),
            precision=_DIST_PREC, preferred_element_type=jnp.float32)
        x_sq = jnp.sum(x * x, axis=1, keepdims=True)
        c_sq = jnp.sum(cb * cb, axis=1)[None, :]
        dist = x_sq + c_sq - 2.0 * xc
        m = jnp.min(dist, axis=1, keepdims=True)
        iota = jax.lax.broadcasted_iota(jnp.int32, dist.shape, 1)
        idx = jnp.min(jnp.where(dist == m, iota, BOOK), axis=1, keepdims=True)
        oh = (iota == idx).astype(jnp.float32)
        oh_ref[:, g, :] = oh
        we = jax.lax.dot_general(
            oh, cb, (((1,), (0,)), ((), ())),
            precision=_GATHER_PREC, preferred_element_type=jnp.float32)
        we_ref[:, g * D:(g + 1) * D] = we
        w_ref[:, g * D:(g + 1) * D] = x + (we - x)


def _vq_call(w_q, codebook, interpret=False):
    grid = (N_CODES // G, BATCH // B_BLK)
    w, w_e, one_hot = pl.pallas_call(
        _vq_body,
        grid=grid,
        in_specs=[
            pl.BlockSpec((B_BLK, G * D), lambda j, i: (i, j)),
            pl.BlockSpec((G, BOOK, D), lambda j, i: (j, 0, 0)),
        ],
        out_specs=[
            pl.BlockSpec((B_BLK, G * D), lambda j, i: (i, j)),
            pl.BlockSpec((B_BLK, G * D), lambda j, i: (i, j)),
            pl.BlockSpec((B_BLK, G, BOOK), lambda j, i: (i, j, 0)),
        ],
        out_shape=[
            jax.ShapeDtypeStruct((BATCH, N_CODES * D), jnp.float32),
            jax.ShapeDtypeStruct((BATCH, N_CODES * D), jnp.float32),
            jax.ShapeDtypeStruct((BATCH, N_CODES, BOOK), jnp.float32),
        ],
        interpret=interpret,
    )(w_q, codebook)
    return w, w_e, one_hot


def kernel(w_q, codebook):
    return _vq_call(w_q, codebook)


# speedup vs baseline: 1.1473x; 1.1473x over previous
import jax
import jax.numpy as jnp
from jax.experimental import pallas as pl

BATCH = 256
N_CODES = 64
BOOK = 1024
D = 64

G = 8
B_BLK = 128

_DIST_PREC = jax.lax.Precision.DEFAULT
_GATHER_PREC = jax.lax.Precision.HIGHEST


def _vq_body(wq_ref, cb_ref, w_ref, we_ref, oh_ref):
    iota2 = jax.lax.broadcasted_iota(jnp.int32, (B_BLK, BOOK), 1)
    idx_cols = []
    for g in range(G):
        x = wq_ref[:, g * D:(g + 1) * D]
        cb = cb_ref[g]
        xc = jax.lax.dot_general(
            x, cb, (((1,), (1,)), ((), ())),
            precision=_DIST_PREC, preferred_element_type=jnp.float32)
        x_sq = jnp.sum(x * x, axis=1, keepdims=True)
        c_sq = jnp.sum(cb * cb, axis=1)[None, :]
        dist = x_sq + c_sq - 2.0 * xc
        m = jnp.min(dist, axis=1, keepdims=True)
        idx = jnp.min(jnp.where(dist == m, iota2, BOOK), axis=1, keepdims=True)
        idx_cols.append(idx)
        oh = (iota2 == idx).astype(jnp.float32)
        we = jax.lax.dot_general(
            oh, cb, (((1,), (0,)), ((), ())),
            precision=_GATHER_PREC, preferred_element_type=jnp.float32)
        we_ref[:, g * D:(g + 1) * D] = we
        w_ref[:, g * D:(g + 1) * D] = x + (we - x)
    idx2 = jnp.concatenate(idx_cols, axis=1)
    iota3 = jax.lax.broadcasted_iota(jnp.int32, (B_BLK, G, BOOK), 2)
    oh_ref[...] = (idx2[:, :, None] == iota3).astype(jnp.float32)


def _vq_call(w_q, codebook, interpret=False):
    grid = (N_CODES // G, BATCH // B_BLK)
    w, w_e, one_hot = pl.pallas_call(
        _vq_body,
        grid=grid,
        in_specs=[
            pl.BlockSpec((B_BLK, G * D), lambda j, i: (i, j)),
            pl.BlockSpec((G, BOOK, D), lambda j, i: (j, 0, 0)),
        ],
        out_specs=[
            pl.BlockSpec((B_BLK, G * D), lambda j, i: (i, j)),
            pl.BlockSpec((B_BLK, G * D), lambda j, i: (i, j)),
            pl.BlockSpec((B_BLK, G, BOOK), lambda j, i: (i, j, 0)),
        ],
        out_shape=[
            jax.ShapeDtypeStruct((BATCH, N_CODES * D), jnp.float32),
            jax.ShapeDtypeStruct((BATCH, N_CODES * D), jnp.float32),
            jax.ShapeDtypeStruct((BATCH, N_CODES, BOOK), jnp.float32),
        ],
        interpret=interpret,
    )(w_q, codebook)
    return w, w_e, one_hot


def kernel(w_q, codebook):
    return _vq_call(w_q, codebook)


# speedup vs baseline: 1.6363x; 1.4261x over previous
import jax
import jax.numpy as jnp
from jax.experimental import pallas as pl

BATCH = 256
N_CODES = 64
BOOK = 1024
D = 64

G = 8
B_BLK = 128

_DIST_PREC = jax.lax.Precision.DEFAULT
_GATHER_PREC = jax.lax.Precision.DEFAULT


def _vq_body(wq_ref, cb_ref, w_ref, we_ref, oh_ref):
    iota2 = jax.lax.broadcasted_iota(jnp.int32, (B_BLK, BOOK), 1)
    idx_cols = []
    for g in range(G):
        x = wq_ref[:, g * D:(g + 1) * D]
        cb = cb_ref[g]
        xc = jax.lax.dot_general(
            x, cb, (((1,), (1,)), ((), ())),
            precision=_DIST_PREC, preferred_element_type=jnp.float32)
        x_sq = jnp.sum(x * x, axis=1, keepdims=True)
        c_sq = jnp.sum(cb * cb, axis=1)[None, :]
        dist = x_sq + c_sq - 2.0 * xc
        m = jnp.min(dist, axis=1, keepdims=True)
        idx = jnp.min(jnp.where(dist == m, iota2, BOOK), axis=1, keepdims=True)
        idx_cols.append(idx)
        oh = (iota2 == idx).astype(jnp.float32)
        we = jax.lax.dot_general(
            oh, cb, (((1,), (0,)), ((), ())),
            precision=_GATHER_PREC, preferred_element_type=jnp.float32)
        we_ref[:, g * D:(g + 1) * D] = we
        w_ref[:, g * D:(g + 1) * D] = x + (we - x)
    idx2 = jnp.concatenate(idx_cols, axis=1)
    iota3 = jax.lax.broadcasted_iota(jnp.int32, (B_BLK, G, BOOK), 2)
    oh_ref[...] = (idx2[:, :, None] == iota3).astype(jnp.float32)


def _vq_call(w_q, codebook, interpret=False):
    grid = (N_CODES // G, BATCH // B_BLK)
    w, w_e, one_hot = pl.pallas_call(
        _vq_body,
        grid=grid,
        in_specs=[
            pl.BlockSpec((B_BLK, G * D), lambda j, i: (i, j)),
            pl.BlockSpec((G, BOOK, D), lambda j, i: (j, 0, 0)),
        ],
        out_specs=[
            pl.BlockSpec((B_BLK, G * D), lambda j, i: (i, j)),
            pl.BlockSpec((B_BLK, G * D), lambda j, i: (i, j)),
            pl.BlockSpec((B_BLK, G, BOOK), lambda j, i: (i, j, 0)),
        ],
        out_shape=[
            jax.ShapeDtypeStruct((BATCH, N_CODES * D), jnp.float32),
            jax.ShapeDtypeStruct((BATCH, N_CODES * D), jnp.float32),
            jax.ShapeDtypeStruct((BATCH, N_CODES, BOOK), jnp.float32),
        ],
        interpret=interpret,
    )(w_q, codebook)
    return w, w_e, one_hot


def kernel(w_q, codebook):
    return _vq_call(w_q, codebook)
